# Initial kernel scaffold; baseline (speedup 1.0000x reference)
import jax
import jax.numpy as jnp
from jax.experimental import pallas as pl


def kernel(x_user, x_transaction, edge_index_pays, edge_index_paid_by, edge_index_linked, Wl_pays, bl_pays, Wr_pays, Wl_paid_by, bl_paid_by, Wr_paid_by, Wl_linked, bl_linked, Wr_linked, W_out, b_out):
    raise NotImplementedError("write your pallas kernel here")


# speedup vs baseline: 6.3918x; 6.3918x over previous
import functools

import jax
import jax.numpy as jnp
from jax import lax
from jax.experimental import pallas as pl
from jax.experimental.pallas import tpu as pltpu
from jax.experimental.pallas import tpu_sc as plsc

N = 10000
D = 128
H = 64
E = 160000
CHUNK = 128
NSUB = 16
E_PAD = 163840
NCHUNKS = E_PAD // CHUNK
CH_PER_TILE = NCHUNKS // NSUB
ACC_ROWS = 10240
ZSLICE = ACC_ROWS // NSUB
WB_ROWS = 624
CNT_ROWS = 10016

_F32 = jnp.float32


def _dot_nt(a, b):
    return lax.dot_general(a, b, (((1,), (1,)), ((), ())),
                           preferred_element_type=_F32)


def _pre_body(xu, xt, wlp, wll, wrp, wrl, hp_o, hl_o, base_o):
    hp_o[...] = _dot_nt(xu[...], wlp[...])
    hl_o[...] = _dot_nt(xt[...], wll[...])
    base_o[...] = _dot_nt(xt[...], wrp[...] + wrl[...])


def _pre(x_user, x_tx, wlp, wll, wrp, wrl):
    blk = 2000
    grid = N // blk
    row_spec = pl.BlockSpec((blk, D), lambda i: (i, 0))
    w_spec = pl.BlockSpec((H, D), lambda i: (0, 0))
    out_spec = pl.BlockSpec((blk, H), lambda i: (i, 0))
    return pl.pallas_call(
        _pre_body,
        grid=(grid,),
        in_specs=[row_spec, row_spec, w_spec, w_spec, w_spec, w_spec],
        out_specs=[out_spec, out_spec, out_spec],
        out_shape=[jax.ShapeDtypeStruct((N, H), _F32)] * 3,
    )(x_user, x_tx, wlp, wll, wrp, wrl)


def _sc_process(h_tbl, src_hbm, dst_hbm, sum_hbm, cnt_hbm,
                idxs, idxd, rows, cntv, acc, sem, s):
    zv = jnp.zeros((16,), _F32)

    def _zc(i, c):
        cntv[pl.ds(i * 16, 16)] = zv
        return c
    lax.fori_loop(0, CNT_ROWS // 16, _zc, 0)

    def _zr(k, c):
        i = k // (H // 16)
        j = k % (H // 16)
        rows[i, pl.ds(j * 16, 16)] = zv
        return c
    lax.fori_loop(0, CHUNK * (H // 16), _zr, 0)
    zbase = pl.multiple_of(s * ZSLICE, 8)
    for k in range(ZSLICE // CHUNK):
        pltpu.sync_copy(rows, acc.at[pl.ds(zbase + k * CHUNK, CHUNK)])
    plsc.subcore_barrier()

    start = pl.multiple_of(CH_PER_TILE * s, 8)
    pltpu.sync_copy(src_hbm.at[pl.ds(start, CH_PER_TILE)], idxs)
    pltpu.sync_copy(dst_hbm.at[pl.ds(start, CH_PER_TILE)], idxd)

    ones = jnp.ones((16,), _F32)

    def _step(g, c):
        pltpu.async_copy(h_tbl.at[idxs.at[g]], rows, sem).wait()
        pltpu.sync_copy(rows, acc.at[idxd.at[g]], add=True)
        for l in range(CHUNK // 16):
            dvec = idxd[g, pl.ds(l * 16, 16)]
            plsc.addupdate_scatter(cntv, [dvec], ones)
        return c

    lax.fori_loop(0, CH_PER_TILE, _step, 0)

    plsc.subcore_barrier()

    wb = pl.multiple_of(s * WB_ROWS, 8)
    pltpu.sync_copy(acc.at[pl.ds(wb, WB_ROWS)],
                    sum_hbm.at[pl.ds(wb, WB_ROWS)])

    @pl.when(s == NSUB - 1)
    def _tail():
        pltpu.sync_copy(acc.at[pl.ds(NSUB * WB_ROWS, N - NSUB * WB_ROWS)],
                        sum_hbm.at[pl.ds(NSUB * WB_ROWS, N - NSUB * WB_ROWS)])

    pltpu.sync_copy(cntv.at[pl.ds(0, N)], cnt_hbm.at[s, 0])


def _sc_body(hp, hl, srcp, dstp, srcl, dstl,
             sum_p, sum_l, cnt_p, cnt_l,
             idxs, idxd, rows, cntv, acc, sem):
    c = lax.axis_index("c")
    s = lax.axis_index("s")

    @pl.when(c == 0)
    def _pays():
        _sc_process(hp, srcp, dstp, sum_p, cnt_p,
                    idxs, idxd, rows, cntv, acc, sem, s)

    @pl.when(c == 1)
    def _linked():
        _sc_process(hl, srcl, dstl, sum_l, cnt_l,
                    idxs, idxd, rows, cntv, acc, sem, s)


def _sc(hp, hl, srcp, dstp, srcl, dstl):
    mesh = plsc.VectorSubcoreMesh(core_axis_name="c", subcore_axis_name="s")
    fn = pl.kernel(
        _sc_body,
        out_type=(
            jax.ShapeDtypeStruct((N, H), _F32),
            jax.ShapeDtypeStruct((N, H), _F32),
            jax.ShapeDtypeStruct((NSUB, 1, N), _F32),
            jax.ShapeDtypeStruct((NSUB, 1, N), _F32),
        ),
        mesh=mesh,
        compiler_params=pltpu.CompilerParams(needs_layout_passes=False,
                                             use_tc_tiling_on_sc=False),
        scratch_types=[
            pltpu.VMEM((CH_PER_TILE, CHUNK), jnp.int32),
            pltpu.VMEM((CH_PER_TILE, CHUNK), jnp.int32),
            pltpu.VMEM((CHUNK, H), _F32),
            pltpu.VMEM((CNT_ROWS,), _F32),
            pltpu.VMEM_SHARED((ACC_ROWS, H), _F32),
            pltpu.SemaphoreType.DMA,
        ],
    )
    return fn(hp, hl, srcp, dstp, srcl, dstl)


def _post_body(sump, suml, cntp, cntl, base, blp, bll, wout, bout,
               txx_o, logit_o):
    cp = jnp.maximum(jnp.sum(cntp[...], axis=0), 1.0)
    cl = jnp.maximum(jnp.sum(cntl[...], axis=0), 1.0)
    h = (sump[...] / cp[:, None] + suml[...] / cl[:, None]
         + base[...] + blp[...] + bll[...]) * 0.5
    h = jnp.maximum(h, 0.0)
    txx_o[...] = h
    logit_o[...] = _dot_nt(h, wout[...]) + bout[0]


def _post(sum_p, sum_l, cnt_p, cnt_l, base, blp, bll, wout, bout):
    vspec = pl.BlockSpec(memory_space=pltpu.VMEM)
    return pl.pallas_call(
        _post_body,
        in_specs=[vspec] * 8 + [pl.BlockSpec(memory_space=pltpu.SMEM)],
        out_shape=[jax.ShapeDtypeStruct((N, H), _F32),
                   jax.ShapeDtypeStruct((N, 8), _F32)],
    )(sum_p, sum_l, cnt_p, cnt_l, base, blp, bll, wout, bout)


def kernel(x_user, x_transaction, edge_index_pays, edge_index_paid_by,
           edge_index_linked, Wl_pays, bl_pays, Wr_pays, Wl_paid_by,
           bl_paid_by, Wr_paid_by, Wl_linked, bl_linked, Wr_linked,
           W_out, b_out):
    del edge_index_paid_by, Wl_paid_by, bl_paid_by, Wr_paid_by

    ei_p = edge_index_pays.astype(jnp.int32)
    ei_l = edge_index_linked.astype(jnp.int32)
    pad = E_PAD - E
    zpad = jnp.zeros((pad,), jnp.int32)
    npad = jnp.full((pad,), N, jnp.int32)
    srcp = jnp.concatenate([ei_p[0], zpad]).reshape(NCHUNKS, CHUNK)
    dstp = jnp.concatenate([ei_p[1], npad]).reshape(NCHUNKS, CHUNK)
    srcl = jnp.concatenate([ei_l[0], zpad]).reshape(NCHUNKS, CHUNK)
    dstl = jnp.concatenate([ei_l[1], npad]).reshape(NCHUNKS, CHUNK)

    hp, hl, base = _pre(x_user, x_transaction, Wl_pays, Wl_linked,
                        Wr_pays, Wr_linked)
    sum_p, sum_l, cnt_p, cnt_l = _sc(hp, hl, srcp, dstp, srcl, dstl)
    cnt_p = cnt_p.reshape(NSUB, N)
    cnt_l = cnt_l.reshape(NSUB, N)
    tx_x, logits = _post(sum_p, sum_l, cnt_p, cnt_l, base,
                         bl_pays.reshape(1, H), bl_linked.reshape(1, H),
                         jnp.pad(W_out, ((0, 7), (0, 0))), b_out)
    return logits[:, 0], tx_x


# speedup vs baseline: 8.0568x; 1.2605x over previous
import functools

import jax
import jax.numpy as jnp
from jax import lax
from jax.experimental import pallas as pl
from jax.experimental.pallas import tpu as pltpu
from jax.experimental.pallas import tpu_sc as plsc

N = 10000
D = 128
H = 64
E = 160000
CHUNK = 128
NSUB = 16
E_PAD = 163840
NCHUNKS = E_PAD // CHUNK
CH_PER_TILE = NCHUNKS // NSUB
ACC_ROWS = 10240
ZSLICE = ACC_ROWS // NSUB
WB_ROWS = 624
CNT_ROWS = 10016
K = 4
NGRP = CH_PER_TILE // (2 * K)

_F32 = jnp.float32


def _dot_nt(a, b):
    return lax.dot_general(a, b, (((1,), (1,)), ((), ())),
                           preferred_element_type=_F32)


def _pre_body(xu, xt, wlp, wll, wrp, wrl, hp_o, hl_o, base_o):
    hp_o[...] = _dot_nt(xu[...], wlp[...])
    hl_o[...] = _dot_nt(xt[...], wll[...])
    base_o[...] = _dot_nt(xt[...], wrp[...] + wrl[...])


def _pre(x_user, x_tx, wlp, wll, wrp, wrl):
    blk = 2000
    grid = N // blk
    row_spec = pl.BlockSpec((blk, D), lambda i: (i, 0))
    w_spec = pl.BlockSpec((H, D), lambda i: (0, 0))
    out_spec = pl.BlockSpec((blk, H), lambda i: (i, 0))
    return pl.pallas_call(
        _pre_body,
        grid=(grid,),
        in_specs=[row_spec, row_spec, w_spec, w_spec, w_spec, w_spec],
        out_specs=[out_spec, out_spec, out_spec],
        out_shape=[jax.ShapeDtypeStruct((N, H), _F32)] * 3,
    )(x_user, x_tx, wlp, wll, wrp, wrl)


def _sc_process(h_tbl, src_hbm, dst_hbm, sum_hbm, cnt_hbm,
                idxs, idxd, rowsA, rowsB, cntv, acc, sem, s):
    rows = rowsA.at[0]
    zv = jnp.zeros((16,), _F32)

    def _zc(i, c):
        cntv[pl.ds(i * 16, 16)] = zv
        return c
    lax.fori_loop(0, CNT_ROWS // 16, _zc, 0)

    def _zr(k, c):
        i = k // (H // 16)
        j = k % (H // 16)
        rows[i, pl.ds(j * 16, 16)] = zv
        return c
    lax.fori_loop(0, CHUNK * (H // 16), _zr, 0)
    zbase = pl.multiple_of(s * ZSLICE, 8)
    for k in range(ZSLICE // CHUNK):
        pltpu.sync_copy(rows, acc.at[pl.ds(zbase + k * CHUNK, CHUNK)])
    plsc.subcore_barrier()

    start = pl.multiple_of(CH_PER_TILE * s, 8)
    ones = jnp.ones((16,), _F32)

    semA, semB, semS, semI = sem

    def _counts(slot, j):
        for l in range(CHUNK // 16):
            dvec = idxd[slot, j, pl.ds(l * 16, 16)]
            plsc.addupdate_scatter(cntv, [dvec], ones)

    def _gather(buf, slot, j0, sm):
        for j in range(K):
            pltpu.async_copy(h_tbl.at[idxs.at[slot, j0 + j]], buf.at[j], sm)

    def _drain_gather(buf, slot, j0, sm):
        for j in range(K):
            pltpu.make_async_copy(h_tbl.at[idxs.at[slot, j0 + j]],
                                  buf.at[j], sm).wait()

    def _scatter(buf, slot, j0):
        for j in range(K):
            _counts(slot, j0 + j)
            pltpu.async_copy(buf.at[j], acc.at[idxd.at[slot, j0 + j]],
                             semS, add=True)
        for j in range(K):
            pltpu.make_async_copy(buf.at[j], acc.at[idxd.at[slot, j0 + j]],
                                  semS).wait()

    def _stage_idx(slot, base, sm):
        pltpu.async_copy(src_hbm.at[pl.ds(start + base, 2 * K)],
                         idxs.at[slot], sm)
        pltpu.async_copy(dst_hbm.at[pl.ds(start + base, 2 * K)],
                         idxd.at[slot], sm)

    def _drain_idx(slot, base, sm):
        pltpu.make_async_copy(src_hbm.at[pl.ds(start + base, 2 * K)],
                              idxs.at[slot], sm).wait()
        pltpu.make_async_copy(dst_hbm.at[pl.ds(start + base, 2 * K)],
                              idxd.at[slot], sm).wait()

    def _pipe(i, c):
        slot = lax.rem(i, 2)
        nxt = 1 - slot
        base = 2 * K * i

        @pl.when(i < NGRP - 1)
        def _prefetch_idx():
            _stage_idx(nxt, base + 2 * K, semI)

        _gather(rowsB, slot, K, semB)
        _drain_gather(rowsA, slot, 0, semA)
        _scatter(rowsA, slot, 0)

        @pl.when(i < NGRP - 1)
        def _next():
            _drain_idx(nxt, base + 2 * K, semI)
            _gather(rowsA, nxt, 0, semA)

        _drain_gather(rowsB, slot, K, semB)
        _scatter(rowsB, slot, K)
        return c

    pltpu.sync_copy(src_hbm.at[pl.ds(start, 2 * K)], idxs.at[0])
    pltpu.sync_copy(dst_hbm.at[pl.ds(start, 2 * K)], idxd.at[0])
    _gather(rowsA, 0, 0, semA)
    lax.fori_loop(0, NGRP, _pipe, 0)

    plsc.subcore_barrier()

    wb = pl.multiple_of(s * WB_ROWS, 8)
    pltpu.sync_copy(acc.at[pl.ds(wb, WB_ROWS)],
                    sum_hbm.at[pl.ds(wb, WB_ROWS)])

    @pl.when(s == NSUB - 1)
    def _tail():
        pltpu.sync_copy(acc.at[pl.ds(NSUB * WB_ROWS, N - NSUB * WB_ROWS)],
                        sum_hbm.at[pl.ds(NSUB * WB_ROWS, N - NSUB * WB_ROWS)])

    pltpu.sync_copy(cntv.at[pl.ds(0, N)], cnt_hbm.at[s, 0])


def _sc_body(hp, hl, srcp, dstp, srcl, dstl,
             sum_p, sum_l, cnt_p, cnt_l,
             idxs, idxd, rowsA, rowsB, cntv, acc, semA, semB, semS, semI):
    c = lax.axis_index("c")
    s = lax.axis_index("s")
    sem = (semA, semB, semS, semI)

    @pl.when(c == 0)
    def _pays():
        _sc_process(hp, srcp, dstp, sum_p, cnt_p,
                    idxs, idxd, rowsA, rowsB, cntv, acc, sem, s)

    @pl.when(c == 1)
    def _linked():
        _sc_process(hl, srcl, dstl, sum_l, cnt_l,
                    idxs, idxd, rowsA, rowsB, cntv, acc, sem, s)


def _sc(hp, hl, srcp, dstp, srcl, dstl):
    mesh = plsc.VectorSubcoreMesh(core_axis_name="c", subcore_axis_name="s")
    fn = pl.kernel(
        _sc_body,
        out_type=(
            jax.ShapeDtypeStruct((N, H), _F32),
            jax.ShapeDtypeStruct((N, H), _F32),
            jax.ShapeDtypeStruct((NSUB, 1, N), _F32),
            jax.ShapeDtypeStruct((NSUB, 1, N), _F32),
        ),
        mesh=mesh,
        compiler_params=pltpu.CompilerParams(needs_layout_passes=False,
                                             use_tc_tiling_on_sc=False),
        scratch_types=[
            pltpu.VMEM((2, 2 * K, CHUNK), jnp.int32),
            pltpu.VMEM((2, 2 * K, CHUNK), jnp.int32),
            pltpu.VMEM((K, CHUNK, H), _F32),
            pltpu.VMEM((K, CHUNK, H), _F32),
            pltpu.VMEM((CNT_ROWS,), _F32),
            pltpu.VMEM_SHARED((ACC_ROWS, H), _F32),
            pltpu.SemaphoreType.DMA,
            pltpu.SemaphoreType.DMA,
            pltpu.SemaphoreType.DMA,
            pltpu.SemaphoreType.DMA,
        ],
    )
    return fn(hp, hl, srcp, dstp, srcl, dstl)


def _post_body(sump, suml, cntp, cntl, base, blp, bll, wout, bout,
               txx_o, logit_o):
    cp = jnp.maximum(jnp.sum(cntp[...], axis=0), 1.0)
    cl = jnp.maximum(jnp.sum(cntl[...], axis=0), 1.0)
    h = (sump[...] / cp[:, None] + suml[...] / cl[:, None]
         + base[...] + blp[...] + bll[...]) * 0.5
    h = jnp.maximum(h, 0.0)
    txx_o[...] = h
    logit_o[...] = _dot_nt(h, wout[...]) + bout[0]


def _post(sum_p, sum_l, cnt_p, cnt_l, base, blp, bll, wout, bout):
    vspec = pl.BlockSpec(memory_space=pltpu.VMEM)
    return pl.pallas_call(
        _post_body,
        in_specs=[vspec] * 8 + [pl.BlockSpec(memory_space=pltpu.SMEM)],
        out_shape=[jax.ShapeDtypeStruct((N, H), _F32),
                   jax.ShapeDtypeStruct((N, 8), _F32)],
    )(sum_p, sum_l, cnt_p, cnt_l, base, blp, bll, wout, bout)


def kernel(x_user, x_transaction, edge_index_pays, edge_index_paid_by,
           edge_index_linked, Wl_pays, bl_pays, Wr_pays, Wl_paid_by,
           bl_paid_by, Wr_paid_by, Wl_linked, bl_linked, Wr_linked,
           W_out, b_out):
    del edge_index_paid_by, Wl_paid_by, bl_paid_by, Wr_paid_by

    ei_p = edge_index_pays.astype(jnp.int32)
    ei_l = edge_index_linked.astype(jnp.int32)
    pad = E_PAD - E
    zpad = jnp.zeros((pad,), jnp.int32)
    npad = jnp.full((pad,), N, jnp.int32)
    srcp = jnp.concatenate([ei_p[0], zpad]).reshape(NCHUNKS, CHUNK)
    dstp = jnp.concatenate([ei_p[1], npad]).reshape(NCHUNKS, CHUNK)
    srcl = jnp.concatenate([ei_l[0], zpad]).reshape(NCHUNKS, CHUNK)
    dstl = jnp.concatenate([ei_l[1], npad]).reshape(NCHUNKS, CHUNK)

    hp, hl, base = _pre(x_user, x_transaction, Wl_pays, Wl_linked,
                        Wr_pays, Wr_linked)
    sum_p, sum_l, cnt_p, cnt_l = _sc(hp, hl, srcp, dstp, srcl, dstl)
    cnt_p = cnt_p.reshape(NSUB, N)
    cnt_l = cnt_l.reshape(NSUB, N)
    tx_x, logits = _post(sum_p, sum_l, cnt_p, cnt_l, base,
                         bl_pays.reshape(1, H), bl_linked.reshape(1, H),
                         jnp.pad(W_out, ((0, 7), (0, 0))), b_out)
    return logits[:, 0], tx_x


# speedup vs baseline: 14.4060x; 1.7881x over previous
import jax
import jax.numpy as jnp
from jax import lax
from jax.experimental import pallas as pl
from jax.experimental.pallas import tpu as pltpu
from jax.experimental.pallas import tpu_sc as plsc

N = 10000
D = 128
H = 64
E = 160000
CHUNK = 128
NCHUNKS = E // CHUNK
NSUB = 16
CH_PER_TILE = 80
K = 4
NGRP = CH_PER_TILE // (2 * K)
TAIL_START = NCHUNKS - 2
ACC_ROWS = 10240
ZSLICE = ACC_ROWS // NSUB
WB_ROWS = 624
CNT_ROWS = 10016

_F32 = jnp.float32


def _dot_nt(a, b):
    return lax.dot_general(a, b, (((1,), (1,)), ((), ())),
                           preferred_element_type=_F32)


def _pre_body(xu, xt, wlp, wll, wrp, wrl, hp_o, hl_o, base_o):
    hp_o[...] = _dot_nt(xu[...], wlp[...])
    hl_o[...] = _dot_nt(xt[...], wll[...])
    base_o[...] = _dot_nt(xt[...], wrp[...] + wrl[...])


def _pre(x_user, x_tx, wlp, wll, wrp, wrl):
    blk = 2000
    grid = N // blk
    row_spec = pl.BlockSpec((blk, D), lambda i: (i, 0))
    w_spec = pl.BlockSpec((H, D), lambda i: (0, 0))
    out_spec = pl.BlockSpec((blk, H), lambda i: (i, 0))
    return pl.pallas_call(
        _pre_body,
        grid=(grid,),
        in_specs=[row_spec, row_spec, w_spec, w_spec, w_spec, w_spec],
        out_specs=[out_spec, out_spec, out_spec],
        out_shape=[jax.ShapeDtypeStruct((N, H), _F32)] * 3,
    )(x_user, x_tx, wlp, wll, wrp, wrl)


def _sc_process(h_tbl, ei_hbm, sum_hbm, cnt_hbm,
                idxs, idxd, rowsA, rowsB, cntv, acc, sem, s):
    src_hbm = ei_hbm.at[0]
    dst_hbm = ei_hbm.at[1]
    semA, semB, semS, semI = sem
    start = pl.multiple_of(CH_PER_TILE * s, 8)
    zv = jnp.zeros((16,), _F32)
    ones = jnp.ones((16,), _F32)

    def _counts(slot, j):
        for l in range(CHUNK // 16):
            dvec = idxd[slot, j, pl.ds(l * 16, 16)]
            plsc.addupdate_scatter(cntv, [dvec], ones)

    def _gather(buf, slot, j0, nj, sm):
        for j in range(nj):
            pltpu.async_copy(h_tbl.at[idxs.at[slot, j0 + j]], buf.at[j], sm)

    def _drain_gather(buf, slot, j0, nj, sm):
        for j in range(nj):
            pltpu.make_async_copy(h_tbl.at[idxs.at[slot, j0 + j]],
                                  buf.at[j], sm).wait()

    def _scatter(buf, slot, j0, nj):
        for j in range(nj):
            _counts(slot, j0 + j)
            pltpu.async_copy(buf.at[j], acc.at[idxd.at[slot, j0 + j]],
                             semS, add=True)
        for j in range(nj):
            pltpu.make_async_copy(buf.at[j], acc.at[idxd.at[slot, j0 + j]],
                                  semS).wait()

    pltpu.sync_copy(src_hbm.at[pl.ds(start, 2 * K)], idxs.at[0])
    pltpu.sync_copy(dst_hbm.at[pl.ds(start, 2 * K)], idxd.at[0])
    _gather(rowsA, 0, 0, K, semA)

    def _zc(i, c):
        cntv[pl.ds(i * 16, 16)] = zv
        return c
    lax.fori_loop(0, CNT_ROWS // 16, _zc, 0)

    zrows = rowsB.at[0]

    def _zr(k, c):
        zrows[k // (H // 16), pl.ds((k % (H // 16)) * 16, 16)] = zv
        return c
    lax.fori_loop(0, CHUNK * (H // 16), _zr, 0)
    zbase = pl.multiple_of(s * ZSLICE, 8)
    for k in range(ZSLICE // CHUNK):
        pltpu.async_copy(zrows, acc.at[pl.ds(zbase + k * CHUNK, CHUNK)],
                         semI)
    for k in range(ZSLICE // CHUNK):
        pltpu.make_async_copy(zrows, acc.at[pl.ds(zbase + k * CHUNK, CHUNK)],
                              semI).wait()
    plsc.subcore_barrier()

    ngrp = jnp.where(s == NSUB - 1, (NCHUNKS - CH_PER_TILE * (NSUB - 1) - 2)
                     // (2 * K), NGRP)

    def _stage_idx(slot, base, sm):
        pltpu.async_copy(src_hbm.at[pl.ds(start + base, 2 * K)],
                         idxs.at[slot], sm)
        pltpu.async_copy(dst_hbm.at[pl.ds(start + base, 2 * K)],
                         idxd.at[slot], sm)

    def _drain_idx(slot, base, sm):
        pltpu.make_async_copy(src_hbm.at[pl.ds(start + base, 2 * K)],
                              idxs.at[slot], sm).wait()
        pltpu.make_async_copy(dst_hbm.at[pl.ds(start + base, 2 * K)],
                              idxd.at[slot], sm).wait()

    def _pipe(i, c):
        slot = lax.rem(i, 2)
        nxt = 1 - slot
        base = 2 * K * i

        @pl.when(i < ngrp - 1)
        def _prefetch_idx():
            _stage_idx(nxt, base + 2 * K, semI)

        _gather(rowsB, slot, K, K, semB)
        _drain_gather(rowsA, slot, 0, K, semA)
        _scatter(rowsA, slot, 0, K)

        @pl.when(i < ngrp - 1)
        def _next():
            _drain_idx(nxt, base + 2 * K, semI)
            _gather(rowsA, nxt, 0, K, semA)

        _drain_gather(rowsB, slot, K, K, semB)
        _scatter(rowsB, slot, K, K)
        return c

    lax.fori_loop(0, ngrp, _pipe, 0)

    @pl.when(s == NSUB - 1)
    def _tail2():
        pltpu.sync_copy(src_hbm.at[pl.ds(TAIL_START, 2)],
                        idxs.at[0, pl.ds(0, 2)])
        pltpu.sync_copy(dst_hbm.at[pl.ds(TAIL_START, 2)],
                        idxd.at[0, pl.ds(0, 2)])
        _gather(rowsA, 0, 0, 2, semA)
        _drain_gather(rowsA, 0, 0, 2, semA)
        _scatter(rowsA, 0, 0, 2)

    plsc.subcore_barrier()

    wb = pl.multiple_of(s * WB_ROWS, 8)
    pltpu.async_copy(acc.at[pl.ds(wb, WB_ROWS)],
                     sum_hbm.at[pl.ds(wb, WB_ROWS)], semA)
    pltpu.async_copy(cntv.at[pl.ds(0, N)], cnt_hbm.at[s, 0], semB)

    @pl.when(s == NSUB - 1)
    def _wb_tail():
        pltpu.sync_copy(acc.at[pl.ds(NSUB * WB_ROWS, N - NSUB * WB_ROWS)],
                        sum_hbm.at[pl.ds(NSUB * WB_ROWS, N - NSUB * WB_ROWS)])

    pltpu.make_async_copy(acc.at[pl.ds(wb, WB_ROWS)],
                          sum_hbm.at[pl.ds(wb, WB_ROWS)], semA).wait()
    pltpu.make_async_copy(cntv.at[pl.ds(0, N)], cnt_hbm.at[s, 0],
                          semB).wait()


def _sc_body(hp, hl, eip, eil,
             sum_p, sum_l, cnt_p, cnt_l,
             idxs, idxd, rowsA, rowsB, cntv, acc, semA, semB, semS, semI):
    c = lax.axis_index("c")
    s = lax.axis_index("s")
    sem = (semA, semB, semS, semI)

    @pl.when(c == 0)
    def _pays():
        _sc_process(hp, eip, sum_p, cnt_p,
                    idxs, idxd, rowsA, rowsB, cntv, acc, sem, s)

    @pl.when(c == 1)
    def _linked():
        _sc_process(hl, eil, sum_l, cnt_l,
                    idxs, idxd, rowsA, rowsB, cntv, acc, sem, s)


def _sc(hp, hl, eip, eil):
    mesh = plsc.VectorSubcoreMesh(core_axis_name="c", subcore_axis_name="s")
    fn = pl.kernel(
        _sc_body,
        out_type=(
            jax.ShapeDtypeStruct((N, H), _F32),
            jax.ShapeDtypeStruct((N, H), _F32),
            jax.ShapeDtypeStruct((NSUB, 1, N), _F32),
            jax.ShapeDtypeStruct((NSUB, 1, N), _F32),
        ),
        mesh=mesh,
        compiler_params=pltpu.CompilerParams(needs_layout_passes=False,
                                             use_tc_tiling_on_sc=False),
        scratch_types=[
            pltpu.VMEM((2, 2 * K, CHUNK), jnp.int32),
            pltpu.VMEM((2, 2 * K, CHUNK), jnp.int32),
            pltpu.VMEM((K, CHUNK, H), _F32),
            pltpu.VMEM((K, CHUNK, H), _F32),
            pltpu.VMEM((CNT_ROWS,), _F32),
            pltpu.VMEM_SHARED((ACC_ROWS, H), _F32),
            pltpu.SemaphoreType.DMA,
            pltpu.SemaphoreType.DMA,
            pltpu.SemaphoreType.DMA,
            pltpu.SemaphoreType.DMA,
        ],
    )
    return fn(hp, hl, eip, eil)


def _post_body(sump, suml, cntp, cntl, base, blp, bll, wout, bout,
               txx_o, logit_o):
    cp = jnp.maximum(jnp.sum(cntp[:, 0, :], axis=0), 1.0)
    cl = jnp.maximum(jnp.sum(cntl[:, 0, :], axis=0), 1.0)
    h = (sump[...] / cp[:, None] + suml[...] / cl[:, None]
         + base[...] + blp[...] + bll[...]) * 0.5
    h = jnp.maximum(h, 0.0)
    txx_o[...] = h
    logit_o[...] = _dot_nt(h, wout[...]) + bout[0]


def _post(sum_p, sum_l, cnt_p, cnt_l, base, blp, bll, wout, bout):
    vspec = pl.BlockSpec(memory_space=pltpu.VMEM)
    return pl.pallas_call(
        _post_body,
        in_specs=[vspec] * 8 + [pl.BlockSpec(memory_space=pltpu.SMEM)],
        out_shape=[jax.ShapeDtypeStruct((N, H), _F32),
                   jax.ShapeDtypeStruct((N, 8), _F32)],
    )(sum_p, sum_l, cnt_p, cnt_l, base, blp, bll, wout, bout)


def kernel(x_user, x_transaction, edge_index_pays, edge_index_paid_by,
           edge_index_linked, Wl_pays, bl_pays, Wr_pays, Wl_paid_by,
           bl_paid_by, Wr_paid_by, Wl_linked, bl_linked, Wr_linked,
           W_out, b_out):
    del edge_index_paid_by, Wl_paid_by, bl_paid_by, Wr_paid_by

    eip = edge_index_pays.astype(jnp.int32).reshape(2, NCHUNKS, CHUNK)
    eil = edge_index_linked.astype(jnp.int32).reshape(2, NCHUNKS, CHUNK)

    hp, hl, base = _pre(x_user, x_transaction, Wl_pays, Wl_linked,
                        Wr_pays, Wr_linked)
    sum_p, sum_l, cnt_p, cnt_l = _sc(hp, hl, eip, eil)
    tx_x, logits = _post(sum_p, sum_l, cnt_p, cnt_l, base,
                         bl_pays.reshape(1, H), bl_linked.reshape(1, H),
                         jnp.pad(W_out, ((0, 7), (0, 0))), b_out)
    return logits[:, 0], tx_x


# speedup vs baseline: 14.5574x; 1.0105x over previous
import jax
import jax.numpy as jnp
from jax import lax
from jax.experimental import pallas as pl
from jax.experimental.pallas import tpu as pltpu
from jax.experimental.pallas import tpu_sc as plsc

N = 10000
D = 128
H = 64
E = 160000
CHUNK = 128
NCHUNKS = E // CHUNK
NSUB = 16
CH_PER_TILE = 80
K = 4
NGRP = CH_PER_TILE // (2 * K)
TAIL_START = NCHUNKS - 2
ACC_ROWS = 10240
ZSLICE = ACC_ROWS // NSUB
WB_ROWS = 624
CNT_ROWS = 10016

_F32 = jnp.float32


def _dot_nt(a, b):
    return lax.dot_general(a, b, (((1,), (1,)), ((), ())),
                           preferred_element_type=_F32)


def _pre_h_body(xu, xt, wlp, wll, hp_o, hl_o):
    hp_o[...] = _dot_nt(xu[...], wlp[...])
    hl_o[...] = _dot_nt(xt[...], wll[...])


def _pre_h(x_user, x_tx, wlp, wll):
    blk = 2000
    grid = N // blk
    row_spec = pl.BlockSpec((blk, D), lambda i: (i, 0))
    w_spec = pl.BlockSpec((H, D), lambda i: (0, 0))
    out_spec = pl.BlockSpec((blk, H), lambda i: (i, 0))
    return pl.pallas_call(
        _pre_h_body,
        grid=(grid,),
        in_specs=[row_spec, row_spec, w_spec, w_spec],
        out_specs=[out_spec, out_spec],
        out_shape=[jax.ShapeDtypeStruct((N, H), _F32)] * 2,
    )(x_user, x_tx, wlp, wll)


def _pre_base_body(xt, wrp, wrl, base_o):
    base_o[...] = _dot_nt(xt[...], wrp[...] + wrl[...])


def _pre_base(x_tx, wrp, wrl):
    blk = 2000
    grid = N // blk
    row_spec = pl.BlockSpec((blk, D), lambda i: (i, 0))
    w_spec = pl.BlockSpec((H, D), lambda i: (0, 0))
    out_spec = pl.BlockSpec((blk, H), lambda i: (i, 0))
    return pl.pallas_call(
        _pre_base_body,
        grid=(grid,),
        in_specs=[row_spec, w_spec, w_spec],
        out_specs=out_spec,
        out_shape=jax.ShapeDtypeStruct((N, H), _F32),
    )(x_tx, wrp, wrl)


def _sc_process(h_tbl, ei_hbm, sum_hbm, cnt_hbm,
                idxs, idxd, rowsA, rowsB, cntv, acc, sem, s):
    src_hbm = ei_hbm.at[0]
    dst_hbm = ei_hbm.at[1]
    semA, semB, semS, semI = sem
    start = pl.multiple_of(CH_PER_TILE * s, 8)
    estart = pl.multiple_of(CH_PER_TILE * CHUNK * s, 8)
    WE = 2 * K * CHUNK
    zv = jnp.zeros((16,), _F32)
    ones = jnp.ones((16,), _F32)

    def _counts(slot, j):
        for l in range(CHUNK // 16):
            dvec = idxd[slot, pl.ds(j * CHUNK + l * 16, 16)]
            plsc.addupdate_scatter(cntv, [dvec], ones)

    def _gather(buf, slot, j0, nj, sm):
        for j in range(nj):
            pltpu.async_copy(
                h_tbl.at[idxs.at[slot, pl.ds((j0 + j) * CHUNK, CHUNK)]],
                buf.at[j], sm)

    def _drain_gather(buf, slot, j0, nj, sm):
        for j in range(nj):
            pltpu.make_async_copy(
                h_tbl.at[idxs.at[slot, pl.ds((j0 + j) * CHUNK, CHUNK)]],
                buf.at[j], sm).wait()

    def _scatter(buf, slot, j0, nj):
        for j in range(nj):
            _counts(slot, j0 + j)
            pltpu.async_copy(
                buf.at[j], acc.at[idxd.at[slot, pl.ds((j0 + j) * CHUNK,
                                                      CHUNK)]],
                semS, add=True)
        for j in range(nj):
            pltpu.make_async_copy(
                buf.at[j], acc.at[idxd.at[slot, pl.ds((j0 + j) * CHUNK,
                                                      CHUNK)]],
                semS).wait()

    pltpu.sync_copy(src_hbm.at[pl.ds(estart, WE)], idxs.at[0])
    pltpu.sync_copy(dst_hbm.at[pl.ds(estart, WE)], idxd.at[0])
    _gather(rowsA, 0, 0, K, semA)

    def _zc(i, c):
        cntv[pl.ds(i * 16, 16)] = zv
        return c
    lax.fori_loop(0, CNT_ROWS // 16, _zc, 0)

    zrows = rowsB.at[0]

    def _zr(k, c):
        zrows[k // (H // 16), pl.ds((k % (H // 16)) * 16, 16)] = zv
        return c
    lax.fori_loop(0, CHUNK * (H // 16), _zr, 0)
    zbase = pl.multiple_of(s * ZSLICE, 8)
    for k in range(ZSLICE // CHUNK):
        pltpu.async_copy(zrows, acc.at[pl.ds(zbase + k * CHUNK, CHUNK)],
                         semI)
    for k in range(ZSLICE // CHUNK):
        pltpu.make_async_copy(zrows, acc.at[pl.ds(zbase + k * CHUNK, CHUNK)],
                              semI).wait()
    plsc.subcore_barrier()

    ngrp = jnp.where(s == NSUB - 1, (NCHUNKS - CH_PER_TILE * (NSUB - 1) - 2)
                     // (2 * K), NGRP)

    def _stage_idx(slot, ebase, sm):
        pltpu.async_copy(src_hbm.at[pl.ds(estart + ebase, WE)],
                         idxs.at[slot], sm)
        pltpu.async_copy(dst_hbm.at[pl.ds(estart + ebase, WE)],
                         idxd.at[slot], sm)

    def _drain_idx(slot, ebase, sm):
        pltpu.make_async_copy(src_hbm.at[pl.ds(estart + ebase, WE)],
                              idxs.at[slot], sm).wait()
        pltpu.make_async_copy(dst_hbm.at[pl.ds(estart + ebase, WE)],
                              idxd.at[slot], sm).wait()

    def _pipe(i, c):
        slot = lax.rem(i, 2)
        nxt = 1 - slot
        base = 2 * K * i

        @pl.when(i < ngrp - 1)
        def _prefetch_idx():
            _stage_idx(nxt, (base + 2 * K) * CHUNK, semI)

        _gather(rowsB, slot, K, K, semB)
        _drain_gather(rowsA, slot, 0, K, semA)
        _scatter(rowsA, slot, 0, K)

        @pl.when(i < ngrp - 1)
        def _next():
            _drain_idx(nxt, (base + 2 * K) * CHUNK, semI)
            _gather(rowsA, nxt, 0, K, semA)

        _drain_gather(rowsB, slot, K, K, semB)
        _scatter(rowsB, slot, K, K)
        return c

    lax.fori_loop(0, ngrp, _pipe, 0)

    @pl.when(s == NSUB - 1)
    def _tail2():
        pltpu.sync_copy(src_hbm.at[pl.ds(TAIL_START * CHUNK, 2 * CHUNK)],
                        idxs.at[0, pl.ds(0, 2 * CHUNK)])
        pltpu.sync_copy(dst_hbm.at[pl.ds(TAIL_START * CHUNK, 2 * CHUNK)],
                        idxd.at[0, pl.ds(0, 2 * CHUNK)])
        _gather(rowsA, 0, 0, 2, semA)
        _drain_gather(rowsA, 0, 0, 2, semA)
        _scatter(rowsA, 0, 0, 2)

    plsc.subcore_barrier()

    wb = pl.multiple_of(s * WB_ROWS, 8)
    pltpu.async_copy(acc.at[pl.ds(wb, WB_ROWS)],
                     sum_hbm.at[pl.ds(wb, WB_ROWS)], semA)
    pltpu.async_copy(cntv.at[pl.ds(0, N)], cnt_hbm.at[s, 0], semB)

    @pl.when(s == NSUB - 1)
    def _wb_tail():
        pltpu.sync_copy(acc.at[pl.ds(NSUB * WB_ROWS, N - NSUB * WB_ROWS)],
                        sum_hbm.at[pl.ds(NSUB * WB_ROWS, N - NSUB * WB_ROWS)])

    pltpu.make_async_copy(acc.at[pl.ds(wb, WB_ROWS)],
                          sum_hbm.at[pl.ds(wb, WB_ROWS)], semA).wait()
    pltpu.make_async_copy(cntv.at[pl.ds(0, N)], cnt_hbm.at[s, 0],
                          semB).wait()


def _sc_body(hp, hl, eip, eil,
             sum_p, sum_l, cnt_p, cnt_l,
             idxs, idxd, rowsA, rowsB, cntv, acc, semA, semB, semS, semI):
    c = lax.axis_index("c")
    s = lax.axis_index("s")
    sem = (semA, semB, semS, semI)

    @pl.when(c == 0)
    def _pays():
        _sc_process(hp, eip, sum_p, cnt_p,
                    idxs, idxd, rowsA, rowsB, cntv, acc, sem, s)

    @pl.when(c == 1)
    def _linked():
        _sc_process(hl, eil, sum_l, cnt_l,
                    idxs, idxd, rowsA, rowsB, cntv, acc, sem, s)


def _sc(hp, hl, eip, eil):
    mesh = plsc.VectorSubcoreMesh(core_axis_name="c", subcore_axis_name="s")
    fn = pl.kernel(
        _sc_body,
        out_type=(
            jax.ShapeDtypeStruct((N, H), _F32),
            jax.ShapeDtypeStruct((N, H), _F32),
            jax.ShapeDtypeStruct((NSUB, 1, N), _F32),
            jax.ShapeDtypeStruct((NSUB, 1, N), _F32),
        ),
        mesh=mesh,
        compiler_params=pltpu.CompilerParams(needs_layout_passes=False,
                                             use_tc_tiling_on_sc=False),
        scratch_types=[
            pltpu.VMEM((2, 2 * K * CHUNK), jnp.int32),
            pltpu.VMEM((2, 2 * K * CHUNK), jnp.int32),
            pltpu.VMEM((K, CHUNK, H), _F32),
            pltpu.VMEM((K, CHUNK, H), _F32),
            pltpu.VMEM((CNT_ROWS,), _F32),
            pltpu.VMEM_SHARED((ACC_ROWS, H), _F32),
            pltpu.SemaphoreType.DMA,
            pltpu.SemaphoreType.DMA,
            pltpu.SemaphoreType.DMA,
            pltpu.SemaphoreType.DMA,
        ],
    )
    return fn(hp, hl, eip, eil)


def _post_body(sump, suml, cntp, cntl, base, blp, bll, wout, bout,
               txx_o, logit_o):
    cp = jnp.maximum(jnp.sum(cntp[:, 0, :], axis=0), 1.0)
    cl = jnp.maximum(jnp.sum(cntl[:, 0, :], axis=0), 1.0)
    h = (sump[...] / cp[:, None] + suml[...] / cl[:, None]
         + base[...] + blp[...] + bll[...]) * 0.5
    h = jnp.maximum(h, 0.0)
    txx_o[...] = h
    logit_o[...] = _dot_nt(h, wout[...]) + bout[0]


def _post(sum_p, sum_l, cnt_p, cnt_l, base, blp, bll, wout, bout):
    vspec = pl.BlockSpec(memory_space=pltpu.VMEM)
    return pl.pallas_call(
        _post_body,
        in_specs=[vspec] * 8 + [pl.BlockSpec(memory_space=pltpu.SMEM)],
        out_shape=[jax.ShapeDtypeStruct((N, H), _F32),
                   jax.ShapeDtypeStruct((N, 8), _F32)],
    )(sum_p, sum_l, cnt_p, cnt_l, base, blp, bll, wout, bout)


def kernel(x_user, x_transaction, edge_index_pays, edge_index_paid_by,
           edge_index_linked, Wl_pays, bl_pays, Wr_pays, Wl_paid_by,
           bl_paid_by, Wr_paid_by, Wl_linked, bl_linked, Wr_linked,
           W_out, b_out):
    del edge_index_paid_by, Wl_paid_by, bl_paid_by, Wr_paid_by

    eip = edge_index_pays.astype(jnp.int32)
    eil = edge_index_linked.astype(jnp.int32)

    hp, hl = _pre_h(x_user, x_transaction, Wl_pays, Wl_linked)
    base = _pre_base(x_transaction, Wr_pays, Wr_linked)
    sum_p, sum_l, cnt_p, cnt_l = _sc(hp, hl, eip, eil)
    tx_x, logits = _post(sum_p, sum_l, cnt_p, cnt_l, base,
                         bl_pays.reshape(1, H), bl_linked.reshape(1, H),
                         jnp.pad(W_out, ((0, 7), (0, 0))), b_out)
    return logits[:, 0], tx_x


# speedup vs baseline: 14.5804x; 1.0016x over previous
import jax
import jax.numpy as jnp
from jax import lax
from jax.experimental import pallas as pl
from jax.experimental.pallas import tpu as pltpu
from jax.experimental.pallas import tpu_sc as plsc

N = 10000
D = 128
H = 64
E = 160000
CHUNK = 128
NCHUNKS = E // CHUNK
NSUB = 16
CH_PER_TILE = 80
K = 4
NGRP = CH_PER_TILE // (2 * K)
TAIL_START = NCHUNKS - 2
ACC_ROWS = 10240
ZSLICE = ACC_ROWS // NSUB
WB_ROWS = 624
CNT_ROWS = 10016

_F32 = jnp.float32


def _dot_nt(a, b):
    return lax.dot_general(a, b, (((1,), (1,)), ((), ())),
                           preferred_element_type=_F32)


def _pre_h_body(xu, xt, wlp, wll, hp_o, hl_o):
    hp_o[...] = _dot_nt(xu[...], wlp[...])
    hl_o[...] = _dot_nt(xt[...], wll[...])


def _pre_h(x_user, x_tx, wlp, wll):
    blk = 2000
    grid = N // blk
    row_spec = pl.BlockSpec((blk, D), lambda i: (i, 0))
    w_spec = pl.BlockSpec((H, D), lambda i: (0, 0))
    out_spec = pl.BlockSpec((blk, H), lambda i: (i, 0))
    return pl.pallas_call(
        _pre_h_body,
        grid=(grid,),
        in_specs=[row_spec, row_spec, w_spec, w_spec],
        out_specs=[out_spec, out_spec],
        out_shape=[jax.ShapeDtypeStruct((N, H), _F32)] * 2,
    )(x_user, x_tx, wlp, wll)


def _pre_base_body(xt, wrp, wrl, base_o):
    base_o[...] = _dot_nt(xt[...], wrp[...] + wrl[...])


def _pre_base(x_tx, wrp, wrl):
    blk = 2000
    grid = N // blk
    row_spec = pl.BlockSpec((blk, D), lambda i: (i, 0))
    w_spec = pl.BlockSpec((H, D), lambda i: (0, 0))
    out_spec = pl.BlockSpec((blk, H), lambda i: (i, 0))
    return pl.pallas_call(
        _pre_base_body,
        grid=(grid,),
        in_specs=[row_spec, w_spec, w_spec],
        out_specs=out_spec,
        out_shape=jax.ShapeDtypeStruct((N, H), _F32),
    )(x_tx, wrp, wrl)


def _sc_process(h_tbl, ei_hbm, sum_hbm, cnt_hbm,
                idxs, idxd, rowsA, rowsB, cntv, acc, sem, s):
    src_hbm = ei_hbm.at[0]
    dst_hbm = ei_hbm.at[1]
    semA, semB, semS, semI = sem
    start = pl.multiple_of(CH_PER_TILE * s, 8)
    estart = pl.multiple_of(CH_PER_TILE * CHUNK * s, 8)
    WE = 2 * K * CHUNK
    zv = jnp.zeros((16,), _F32)
    ones = jnp.ones((16,), _F32)

    def _counts(slot, j):
        for l in range(CHUNK // 16):
            dvec = idxd[slot, pl.ds(j * CHUNK + l * 16, 16)]
            plsc.addupdate_scatter(cntv, [dvec], ones)

    def _gather(buf, slot, j0, nj, sm):
        for j in range(nj):
            pltpu.async_copy(
                h_tbl.at[idxs.at[slot, pl.ds((j0 + j) * CHUNK, CHUNK)]],
                buf.at[j], sm)

    def _drain_gather(buf, slot, j0, nj, sm):
        for j in range(nj):
            pltpu.make_async_copy(
                h_tbl.at[idxs.at[slot, pl.ds((j0 + j) * CHUNK, CHUNK)]],
                buf.at[j], sm).wait()

    def _scatter(buf, slot, j0, nj):
        for j in range(nj):
            pltpu.async_copy(
                buf.at[j], acc.at[idxd.at[slot, pl.ds((j0 + j) * CHUNK,
                                                      CHUNK)]],
                semS, add=True)
        for j in range(nj):
            _counts(slot, j0 + j)
        for j in range(nj):
            pltpu.make_async_copy(
                buf.at[j], acc.at[idxd.at[slot, pl.ds((j0 + j) * CHUNK,
                                                      CHUNK)]],
                semS).wait()

    pltpu.sync_copy(src_hbm.at[pl.ds(estart, WE)], idxs.at[0])
    pltpu.sync_copy(dst_hbm.at[pl.ds(estart, WE)], idxd.at[0])
    _gather(rowsA, 0, 0, K, semA)

    def _zc(i, c):
        cntv[pl.ds(i * 16, 16)] = zv
        return c
    lax.fori_loop(0, CNT_ROWS // 16, _zc, 0)

    zrows = rowsB.at[0]

    def _zr(k, c):
        zrows[k // (H // 16), pl.ds((k % (H // 16)) * 16, 16)] = zv
        return c
    lax.fori_loop(0, CHUNK * (H // 16), _zr, 0)
    zbase = pl.multiple_of(s * ZSLICE, 8)
    for k in range(ZSLICE // CHUNK):
        pltpu.async_copy(zrows, acc.at[pl.ds(zbase + k * CHUNK, CHUNK)],
                         semI)
    for k in range(ZSLICE // CHUNK):
        pltpu.make_async_copy(zrows, acc.at[pl.ds(zbase + k * CHUNK, CHUNK)],
                              semI).wait()
    plsc.subcore_barrier()

    ngrp = jnp.where(s == NSUB - 1, (NCHUNKS - CH_PER_TILE * (NSUB - 1) - 2)
                     // (2 * K), NGRP)

    def _stage_idx(slot, ebase, sm):
        pltpu.async_copy(src_hbm.at[pl.ds(estart + ebase, WE)],
                         idxs.at[slot], sm)
        pltpu.async_copy(dst_hbm.at[pl.ds(estart + ebase, WE)],
                         idxd.at[slot], sm)

    def _drain_idx(slot, ebase, sm):
        pltpu.make_async_copy(src_hbm.at[pl.ds(estart + ebase, WE)],
                              idxs.at[slot], sm).wait()
        pltpu.make_async_copy(dst_hbm.at[pl.ds(estart + ebase, WE)],
                              idxd.at[slot], sm).wait()

    def _pipe(i, c):
        slot = lax.rem(i, 2)
        nxt = 1 - slot
        base = 2 * K * i

        @pl.when(i < ngrp - 1)
        def _prefetch_idx():
            _stage_idx(nxt, (base + 2 * K) * CHUNK, semI)

        _gather(rowsB, slot, K, K, semB)
        _drain_gather(rowsA, slot, 0, K, semA)
        _scatter(rowsA, slot, 0, K)

        @pl.when(i < ngrp - 1)
        def _next():
            _drain_idx(nxt, (base + 2 * K) * CHUNK, semI)
            _gather(rowsA, nxt, 0, K, semA)

        _drain_gather(rowsB, slot, K, K, semB)
        _scatter(rowsB, slot, K, K)
        return c

    lax.fori_loop(0, ngrp, _pipe, 0)

    @pl.when(s == NSUB - 1)
    def _tail2():
        pltpu.sync_copy(src_hbm.at[pl.ds(TAIL_START * CHUNK, 2 * CHUNK)],
                        idxs.at[0, pl.ds(0, 2 * CHUNK)])
        pltpu.sync_copy(dst_hbm.at[pl.ds(TAIL_START * CHUNK, 2 * CHUNK)],
                        idxd.at[0, pl.ds(0, 2 * CHUNK)])
        _gather(rowsA, 0, 0, 2, semA)
        _drain_gather(rowsA, 0, 0, 2, semA)
        _scatter(rowsA, 0, 0, 2)

    plsc.subcore_barrier()

    wb = pl.multiple_of(s * WB_ROWS, 8)
    pltpu.async_copy(acc.at[pl.ds(wb, WB_ROWS)],
                     sum_hbm.at[pl.ds(wb, WB_ROWS)], semA)
    pltpu.async_copy(cntv.at[pl.ds(0, N)], cnt_hbm.at[s, 0], semB)

    @pl.when(s == NSUB - 1)
    def _wb_tail():
        pltpu.sync_copy(acc.at[pl.ds(NSUB * WB_ROWS, N - NSUB * WB_ROWS)],
                        sum_hbm.at[pl.ds(NSUB * WB_ROWS, N - NSUB * WB_ROWS)])

    pltpu.make_async_copy(acc.at[pl.ds(wb, WB_ROWS)],
                          sum_hbm.at[pl.ds(wb, WB_ROWS)], semA).wait()
    pltpu.make_async_copy(cntv.at[pl.ds(0, N)], cnt_hbm.at[s, 0],
                          semB).wait()


def _sc_body(hp, hl, eip, eil,
             sum_p, sum_l, cnt_p, cnt_l,
             idxs, idxd, rowsA, rowsB, cntv, acc, semA, semB, semS, semI):
    c = lax.axis_index("c")
    s = lax.axis_index("s")
    sem = (semA, semB, semS, semI)

    @pl.when(c == 0)
    def _pays():
        _sc_process(hp, eip, sum_p, cnt_p,
                    idxs, idxd, rowsA, rowsB, cntv, acc, sem, s)

    @pl.when(c == 1)
    def _linked():
        _sc_process(hl, eil, sum_l, cnt_l,
                    idxs, idxd, rowsA, rowsB, cntv, acc, sem, s)


def _sc(hp, hl, eip, eil):
    mesh = plsc.VectorSubcoreMesh(core_axis_name="c", subcore_axis_name="s")
    fn = pl.kernel(
        _sc_body,
        out_type=(
            jax.ShapeDtypeStruct((N, H), _F32),
            jax.ShapeDtypeStruct((N, H), _F32),
            jax.ShapeDtypeStruct((NSUB, 1, N), _F32),
            jax.ShapeDtypeStruct((NSUB, 1, N), _F32),
        ),
        mesh=mesh,
        compiler_params=pltpu.CompilerParams(needs_layout_passes=False,
                                             use_tc_tiling_on_sc=False),
        scratch_types=[
            pltpu.VMEM((2, 2 * K * CHUNK), jnp.int32),
            pltpu.VMEM((2, 2 * K * CHUNK), jnp.int32),
            pltpu.VMEM((K, CHUNK, H), _F32),
            pltpu.VMEM((K, CHUNK, H), _F32),
            pltpu.VMEM((CNT_ROWS,), _F32),
            pltpu.VMEM_SHARED((ACC_ROWS, H), _F32),
            pltpu.SemaphoreType.DMA,
            pltpu.SemaphoreType.DMA,
            pltpu.SemaphoreType.DMA,
            pltpu.SemaphoreType.DMA,
        ],
    )
    return fn(hp, hl, eip, eil)


def _post_body(sump, suml, cntp, cntl, base, blp, bll, wout, bout,
               txx_o, logit_o):
    cp = jnp.maximum(jnp.sum(cntp[:, 0, :], axis=0), 1.0)
    cl = jnp.maximum(jnp.sum(cntl[:, 0, :], axis=0), 1.0)
    h = (sump[...] / cp[:, None] + suml[...] / cl[:, None]
         + base[...] + blp[...] + bll[...]) * 0.5
    h = jnp.maximum(h, 0.0)
    txx_o[...] = h
    logit_o[...] = _dot_nt(h, wout[...]) + bout[0]


def _post(sum_p, sum_l, cnt_p, cnt_l, base, blp, bll, wout, bout):
    vspec = pl.BlockSpec(memory_space=pltpu.VMEM)
    return pl.pallas_call(
        _post_body,
        in_specs=[vspec] * 8 + [pl.BlockSpec(memory_space=pltpu.SMEM)],
        out_shape=[jax.ShapeDtypeStruct((N, H), _F32),
                   jax.ShapeDtypeStruct((N, 8), _F32)],
    )(sum_p, sum_l, cnt_p, cnt_l, base, blp, bll, wout, bout)


def kernel(x_user, x_transaction, edge_index_pays, edge_index_paid_by,
           edge_index_linked, Wl_pays, bl_pays, Wr_pays, Wl_paid_by,
           bl_paid_by, Wr_paid_by, Wl_linked, bl_linked, Wr_linked,
           W_out, b_out):
    del edge_index_paid_by, Wl_paid_by, bl_paid_by, Wr_paid_by

    eip = edge_index_pays.astype(jnp.int32)
    eil = edge_index_linked.astype(jnp.int32)

    hp, hl = _pre_h(x_user, x_transaction, Wl_pays, Wl_linked)
    base = _pre_base(x_transaction, Wr_pays, Wr_linked)
    sum_p, sum_l, cnt_p, cnt_l = _sc(hp, hl, eip, eil)
    tx_x, logits = _post(sum_p, sum_l, cnt_p, cnt_l, base,
                         bl_pays.reshape(1, H), bl_linked.reshape(1, H),
                         jnp.pad(W_out, ((0, 7), (0, 0))), b_out)
    return logits[:, 0], tx_x
